# speedup vs baseline: 6.0809x; 6.0809x over previous
import functools

import jax
import jax.numpy as jnp
from jax import lax
from jax.experimental import pallas as pl
from jax.experimental.pallas import tpu as pltpu
from jax.experimental.pallas import tpu_sc as plsc

B, P, D, OUT = 4096, 32, 128, 128
LANES = 16
NC, NS = 2, 16
NW = NC * NS
EPW = B // NW
CB = 8
NCHUNKS = EPW // CB
DBLK = D // LANES


def _sc_segment_sum(x_flat):
  mesh = plsc.VectorSubcoreMesh(core_axis_name="c", subcore_axis_name="s")

  @functools.partial(
      pl.kernel,
      out_type=jax.ShapeDtypeStruct((B, D), jnp.float32),
      mesh=mesh,
      scratch_types=[
          pltpu.VMEM((CB * P, D), jnp.float32),
          pltpu.VMEM((CB, D), jnp.float32),
      ],
  )
  def k(x_hbm, out_hbm, buf, obuf):
    wid = lax.axis_index("s") * NC + lax.axis_index("c")
    elem0 = wid * EPW

    def chunk_body(c, carry):
      base_e = elem0 + c * CB
      pltpu.sync_copy(x_hbm.at[pl.ds(base_e * P, CB * P)], buf)

      def elem_body(e, carry2):
        def p_body(p, accs):
          row = e * P + p
          return tuple(
              accs[d] + buf[row, pl.ds(d * LANES, LANES)]
              for d in range(DBLK)
          )

        accs = lax.fori_loop(
            0, P, p_body,
            tuple(jnp.zeros((LANES,), jnp.float32) for _ in range(DBLK)),
        )
        for d in range(DBLK):
          obuf[e, pl.ds(d * LANES, LANES)] = accs[d]
        return carry2

      lax.fori_loop(0, CB, elem_body, 0)
      pltpu.sync_copy(obuf, out_hbm.at[pl.ds(base_e, CB)])
      return carry

    lax.fori_loop(0, NCHUNKS, chunk_body, 0)

  return k(x_flat)


def _tc_linear(s, w_t, pb):
  def mm(s_ref, wt_ref, pb_ref, o_ref):
    o_ref[...] = (
        jnp.dot(s_ref[...], wt_ref[...], preferred_element_type=jnp.float32)
        + pb_ref[...]
    )

  return pl.pallas_call(
      mm,
      out_shape=jax.ShapeDtypeStruct((B, OUT), jnp.float32),
  )(s, w_t, pb)


@jax.jit
def kernel(pert_batch, W, b):
  x_flat = pert_batch.reshape(B * P, D)
  s = _sc_segment_sum(x_flat)
  return _tc_linear(s, W.T, (P * b).reshape(1, OUT))


# speedup vs baseline: 6.7035x; 1.1024x over previous
import functools

import jax
import jax.numpy as jnp
from jax import lax
from jax.experimental import pallas as pl
from jax.experimental.pallas import tpu as pltpu
from jax.experimental.pallas import tpu_sc as plsc

B, P, D, OUT = 4096, 32, 128, 128
LANES = 16
NC, NS = 2, 16
NW = NC * NS
EPW = B // NW
CB = 8
NCHUNKS = EPW // CB
DBLK = D // LANES


def _sc_segment_sum(x_flat):
  mesh = plsc.VectorSubcoreMesh(core_axis_name="c", subcore_axis_name="s")

  @functools.partial(
      pl.kernel,
      out_type=jax.ShapeDtypeStruct((B, D), jnp.float32),
      mesh=mesh,
      scratch_types=[
          pltpu.VMEM((CB * P, D), jnp.float32),
          pltpu.VMEM((CB * P, D), jnp.float32),
          pltpu.VMEM((EPW, D), jnp.float32),
          pltpu.SemaphoreType.DMA,
          pltpu.SemaphoreType.DMA,
      ],
  )
  def k(x_hbm, out_hbm, buf0, buf1, stage, sem0, sem1):
    wid = lax.axis_index("s") * NC + lax.axis_index("c")
    elem0 = wid * EPW
    bufs = (buf0, buf1)
    sems = (sem0, sem1)

    def fetch(c, par):
      pltpu.async_copy(
          x_hbm.at[pl.ds((elem0 + c * CB) * P, CB * P)], bufs[par], sems[par]
      )

    def reduce_chunk(c, par):
      buf = bufs[par]

      def elem_body(e, carry):
        base = e * P
        for d in range(DBLK):
          dsl = pl.ds(d * LANES, LANES)
          acc = buf[base, dsl]
          for p in range(1, P):
            acc = acc + buf[base + p, dsl]
          stage[c * CB + e, dsl] = acc
        return carry

      lax.fori_loop(0, CB, elem_body, 0)

    def wait(c, par):
      pltpu.make_async_copy(
          x_hbm.at[pl.ds((elem0 + c * CB) * P, CB * P)], bufs[par], sems[par]
      ).wait()

    fetch(0, 0)
    fetch(1, 1)

    @pl.loop(0, NCHUNKS, step=2)
    def _ring(g):
      wait(g, 0)
      reduce_chunk(g, 0)
      pl.when(g + 2 < NCHUNKS)(lambda: fetch(g + 2, 0))
      wait(g + 1, 1)
      reduce_chunk(g + 1, 1)
      pl.when(g + 3 < NCHUNKS)(lambda: fetch(g + 3, 1))

    pltpu.sync_copy(stage, out_hbm.at[pl.ds(elem0, EPW)])

  return k(x_flat)


def _tc_linear(s, w_t, pb):
  def mm(s_ref, wt_ref, pb_ref, o_ref):
    o_ref[...] = (
        jnp.dot(s_ref[...], wt_ref[...], preferred_element_type=jnp.float32)
        + pb_ref[...]
    )

  return pl.pallas_call(
      mm,
      out_shape=jax.ShapeDtypeStruct((B, OUT), jnp.float32),
  )(s, w_t, pb)


@jax.jit
def kernel(pert_batch, W, b):
  x_flat = pert_batch.reshape(B * P, D)
  s = _sc_segment_sum(x_flat)
  return _tc_linear(s, W.T, (P * b).reshape(1, OUT))


# speedup vs baseline: 8.6176x; 1.2855x over previous
import functools

import jax
import jax.numpy as jnp
from jax import lax
from jax.experimental import pallas as pl
from jax.experimental.pallas import tpu as pltpu
from jax.experimental.pallas import tpu_sc as plsc

B, P, D, OUT = 4096, 32, 128, 128
LANES = 16
NC, NS = 2, 16
NW = NC * NS
EPW = B // NW
CB = 8
NCHUNKS = EPW // CB
DBLK = D // LANES


def _sc_segment_sum(x_flat):
  mesh = plsc.VectorSubcoreMesh(core_axis_name="c", subcore_axis_name="s")

  @functools.partial(
      pl.kernel,
      out_type=jax.ShapeDtypeStruct((B, D), jnp.float32),
      mesh=mesh,
      scratch_types=[
          pltpu.VMEM((CB * P, D), jnp.float32),
          pltpu.VMEM((CB * P, D), jnp.float32),
          pltpu.VMEM((EPW, D), jnp.float32),
          pltpu.SemaphoreType.DMA,
          pltpu.SemaphoreType.DMA,
      ],
  )
  def k(x_hbm, out_hbm, buf0, buf1, stage, sem0, sem1):
    wid = lax.axis_index("s") * NC + lax.axis_index("c")
    elem0 = wid * EPW
    bufs = (buf0, buf1)
    sems = (sem0, sem1)

    def fetch(c, par):
      pltpu.async_copy(
          x_hbm.at[pl.ds((elem0 + c * CB) * P, CB * P)], bufs[par], sems[par]
      )

    def reduce_chunk(c, par):
      buf = bufs[par]

      def elem_body(e, carry):
        base = e * P
        dsls = [pl.ds(d * LANES, LANES) for d in range(DBLK)]
        row_out = c * CB + e
        for d0 in range(0, DBLK, 4):
          grp = dsls[d0:d0 + 4]
          accs = [buf[base, dsl] for dsl in grp]
          for p in range(1, P):
            for i, dsl in enumerate(grp):
              accs[i] = accs[i] + buf[base + p, dsl]
          for i, dsl in enumerate(grp):
            stage[row_out, dsl] = accs[i]
        return carry

      lax.fori_loop(0, CB, elem_body, 0)

    def wait(c, par):
      pltpu.make_async_copy(
          x_hbm.at[pl.ds((elem0 + c * CB) * P, CB * P)], bufs[par], sems[par]
      ).wait()

    fetch(0, 0)
    fetch(1, 1)

    @pl.loop(0, NCHUNKS, step=2)
    def _ring(g):
      wait(g, 0)
      reduce_chunk(g, 0)
      pl.when(g + 2 < NCHUNKS)(lambda: fetch(g + 2, 0))
      wait(g + 1, 1)
      reduce_chunk(g + 1, 1)
      pl.when(g + 3 < NCHUNKS)(lambda: fetch(g + 3, 1))

    pltpu.sync_copy(stage, out_hbm.at[pl.ds(elem0, EPW)])

  return k(x_flat)


def _tc_linear(s, w_t, pb):
  def mm(s_ref, wt_ref, pb_ref, o_ref):
    o_ref[...] = (
        jnp.dot(s_ref[...], wt_ref[...], preferred_element_type=jnp.float32)
        + pb_ref[...]
    )

  return pl.pallas_call(
      mm,
      out_shape=jax.ShapeDtypeStruct((B, OUT), jnp.float32),
  )(s, w_t, pb)


@jax.jit
def kernel(pert_batch, W, b):
  x_flat = pert_batch.reshape(B * P, D)
  s = _sc_segment_sum(x_flat)
  return _tc_linear(s, W.T, (P * b).reshape(1, OUT))
